# Initial kernel scaffold; baseline (speedup 1.0000x reference)
import jax
import jax.numpy as jnp
from jax.experimental import pallas as pl


def kernel(x, edge_index, edge_length, edge_attr, W1, mlp_w1, mlp_b1, mlp_w2, mlp_b2, W2, b2, Wlin, blin):
    raise NotImplementedError("write your pallas kernel here")


# speedup vs baseline: 1.3987x; 1.3987x over previous
import functools

import jax
import jax.numpy as jnp
from jax import lax
from jax.experimental import pallas as pl
from jax.experimental.pallas import tpu as pltpu
from jax.experimental.pallas import tpu_sc as plsc

N = 10000
E = 320000
H = 128
G = 50
F = 128
CUTOFF = 10.0

NC = 2
NS = 16
NW = NC * NS
EPW = E // NW
K = 80
STEPS = EPW // K
RPT = 640
NPAD = NS * RPT

_LOG2 = 0.6931471805599453


def _ssp(v):
    return jnp.maximum(v, 0.0) + jnp.log1p(jnp.exp(-jnp.abs(v))) - _LOG2


def _h_body(x_ref, w_ref, o_ref):
    o_ref[...] = jnp.dot(x_ref[...], w_ref[...], preferred_element_type=jnp.float32)


def _compute_h(x, w1t):
    bn = 2000
    return pl.pallas_call(
        _h_body,
        grid=(N // bn,),
        in_specs=[
            pl.BlockSpec((bn, H), lambda i: (i, 0)),
            pl.BlockSpec((H, F), lambda i: (0, 0)),
        ],
        out_specs=pl.BlockSpec((bn, F), lambda i: (i, 0)),
        out_shape=jax.ShapeDtypeStruct((N, F), jnp.float32),
    )(x, w1t)


def _wf_body(attr_ref, len_ref, w1_ref, b1_ref, w2_ref, b2_ref, o_ref):
    a = attr_ref[...]
    hid = jnp.dot(a, w1_ref[...], preferred_element_type=jnp.float32) + b1_ref[...][None, :]
    hid = _ssp(hid)
    wf = jnp.dot(hid, w2_ref[...], preferred_element_type=jnp.float32) + b2_ref[...][None, :]
    el = len_ref[...]
    c = 0.5 * (jnp.cos(el * (jnp.pi / CUTOFF)) + 1.0)
    c = c * (el <= CUTOFF).astype(jnp.float32) * (el >= 0.0).astype(jnp.float32)
    o_ref[...] = wf * c


def _compute_wfilt(edge_attr, edge_len2, mw1t, mb1, mw2t, mb2):
    be = 2000
    return pl.pallas_call(
        _wf_body,
        grid=(E // be,),
        in_specs=[
            pl.BlockSpec((be, G), lambda i: (i, 0)),
            pl.BlockSpec((be, 1), lambda i: (i, 0)),
            pl.BlockSpec((G, F), lambda i: (0, 0)),
            pl.BlockSpec((F,), lambda i: (0,)),
            pl.BlockSpec((F, F), lambda i: (0, 0)),
            pl.BlockSpec((F,), lambda i: (0,)),
        ],
        out_specs=pl.BlockSpec((be, F), lambda i: (i, 0)),
        out_shape=jax.ShapeDtypeStruct((E, F), jnp.float32),
    )(edge_attr, edge_len2, mw1t, mb1, mw2t, mb2)


def _sc_body(h_hbm, src_hbm, dst_hbm, wf_hbm, z_hbm,
             mij_hbm, part_hbm,
             srcv, dstv, rows, wfv, acc, sem):
    cid = lax.axis_index("c")
    sid = lax.axis_index("s")
    wid = sid * NC + cid

    pltpu.sync_copy(z_hbm, acc.at[pl.ds(sid * RPT, RPT)])
    plsc.subcore_barrier()

    e0 = wid * EPW

    def step(t, carry):
        base = e0 + t * K
        pltpu.sync_copy(src_hbm.at[wid, t], srcv)
        pltpu.sync_copy(dst_hbm.at[wid, t], dstv)
        cp = pltpu.async_copy(h_hbm.at[srcv], rows, sem)
        pltpu.sync_copy(wf_hbm.at[pl.ds(base, K)], wfv)
        cp.wait()

        def mulrow(r, c2):
            for c in range(F // 16):
                rows[r, pl.ds(c * 16, 16)] = (
                    rows[r, pl.ds(c * 16, 16)] * wfv[r, pl.ds(c * 16, 16)]
                )
            return c2

        lax.fori_loop(0, K, mulrow, 0)
        pltpu.sync_copy(rows, mij_hbm.at[pl.ds(base, K)])
        pltpu.sync_copy(rows, acc.at[dstv], add=True)
        return carry

    lax.fori_loop(0, STEPS, step, 0)
    plsc.subcore_barrier()
    pltpu.sync_copy(acc.at[pl.ds(sid * RPT, RPT)],
                    part_hbm.at[pl.ds(cid * NPAD + sid * RPT, RPT)])


@functools.cache
def _get_sc_call():
    return pl.kernel(
        _sc_body,
        out_type=(
            jax.ShapeDtypeStruct((E, F), jnp.float32),
            jax.ShapeDtypeStruct((2 * NPAD, F), jnp.float32),
        ),
        mesh=plsc.VectorSubcoreMesh(core_axis_name="c", subcore_axis_name="s"),
        scratch_types=[
            pltpu.VMEM((K,), jnp.int32),
            pltpu.VMEM((K,), jnp.int32),
            pltpu.VMEM((K, F), jnp.float32),
            pltpu.VMEM((K, F), jnp.float32),
            pltpu.VMEM_SHARED((NPAD, F), jnp.float32),
            pltpu.SemaphoreType.DMA,
        ],
    )


def _out_body(x_ref, p0_ref, p1_ref, w2_ref, b2_ref, wlx_ref, wlm_ref, bl_ref, o_ref):
    m = p0_ref[...] + p1_ref[...]
    t = jnp.dot(m, w2_ref[...], preferred_element_type=jnp.float32) + b2_ref[...][None, :]
    t = _ssp(t)
    o_ref[...] = (
        jnp.dot(x_ref[...], wlx_ref[...], preferred_element_type=jnp.float32)
        + jnp.dot(t, wlm_ref[...], preferred_element_type=jnp.float32)
        + bl_ref[...][None, :]
    )


def _compute_out(x, p0, p1, w2t, b2, wlxt, wlmt, blin):
    bn = 2000
    return pl.pallas_call(
        _out_body,
        grid=(N // bn,),
        in_specs=[
            pl.BlockSpec((bn, H), lambda i: (i, 0)),
            pl.BlockSpec((bn, F), lambda i: (i, 0)),
            pl.BlockSpec((bn, F), lambda i: (i, 0)),
            pl.BlockSpec((F, H), lambda i: (0, 0)),
            pl.BlockSpec((H,), lambda i: (0,)),
            pl.BlockSpec((H, H), lambda i: (0, 0)),
            pl.BlockSpec((H, H), lambda i: (0, 0)),
            pl.BlockSpec((H,), lambda i: (0,)),
        ],
        out_specs=pl.BlockSpec((bn, H), lambda i: (i, 0)),
        out_shape=jax.ShapeDtypeStruct((N, H), jnp.float32),
    )(x, p0, p1, w2t, b2, wlxt, wlmt, blin)


def kernel(x, edge_index, edge_length, edge_attr,
           W1, mlp_w1, mlp_b1, mlp_w2, mlp_b2, W2, b2, Wlin, blin):
    h = _compute_h(x, W1.T)
    wf = _compute_wfilt(edge_attr, edge_length.reshape(E, 1),
                        mlp_w1.T, mlp_b1, mlp_w2.T, mlp_b2)
    src2 = edge_index[0].reshape(NW, STEPS, K)
    dst2 = edge_index[1].reshape(NW, STEPS, K)
    z = jnp.zeros((RPT, F), jnp.float32)
    mij, part = _get_sc_call()(h, src2, dst2, wf, z)
    out = _compute_out(x, part[:N], part[NPAD:NPAD + N], W2.T, b2,
                       Wlin[:, :H].T, Wlin[:, H:].T, blin)
    return out, mij
